# baseline (speedup 1.0000x reference)
import functools

import jax
import jax.numpy as jnp
from jax import lax
from jax.experimental import pallas as pl
from jax.experimental.pallas import tpu as pltpu
from jax.experimental.pallas import tpu_sc as plsc

_B, _S, _D = 4, 8192, 1024
_NC, _NS = 2, 16
_NW = _NC * _NS
_SPW = _S // _NW
_C = 32
_NCH = _SPW // _C
_CW = _C * _D

_mesh = plsc.VectorSubcoreMesh(core_axis_name="c", subcore_axis_name="s")


@functools.partial(
    pl.kernel,
    out_type=jax.ShapeDtypeStruct((_B * _S * _D,), jnp.float32),
    mesh=_mesh,
    scratch_types=[
        pltpu.VMEM((_CW,), jnp.float32),
        pltpu.VMEM((_CW,), jnp.float32),
    ],
)
def _pos_add(x_hbm, tab_hbm, out_hbm, buf_p, buf_x):
    wid = lax.axis_index("s") * _NC + lax.axis_index("c")
    s_word0 = wid * (_SPW * _D)

    def chunk_body(c, carry):
        t_off = s_word0 + c * _CW
        pltpu.sync_copy(tab_hbm.at[pl.ds(t_off, _CW)], buf_p)

        def batch_body(b, carry2):
            off = b * (_S * _D) + t_off
            pltpu.sync_copy(x_hbm.at[pl.ds(off, _CW)], buf_x)

            @plsc.parallel_loop(0, _CW, step=16, unroll=8)
            def add_body(i):
                buf_x[pl.ds(i, 16)] = buf_x[pl.ds(i, 16)] + buf_p[pl.ds(i, 16)]

            pltpu.sync_copy(buf_x, out_hbm.at[pl.ds(off, _CW)])
            return carry2

        lax.fori_loop(0, _B, batch_body, 0)
        return carry

    lax.fori_loop(0, _NCH, chunk_body, 0)


def kernel(x, pos_table):
    out = _pos_add(x.reshape(_B * _S * _D), pos_table.reshape(_S * _D))
    return out.reshape(_B, _S, _D)


# speedup vs baseline: 1.1889x; 1.1889x over previous
import functools

import jax
import jax.numpy as jnp
from jax import lax
from jax.experimental import pallas as pl
from jax.experimental.pallas import tpu as pltpu
from jax.experimental.pallas import tpu_sc as plsc

_B, _S, _D = 4, 8192, 1024
_NC, _NS = 2, 16
_NW = _NC * _NS
_SPW = _S // _NW
_C = 16
_NCH = _SPW // _C
_CW = _C * _D
_SD = _S * _D

_mesh = plsc.VectorSubcoreMesh(core_axis_name="c", subcore_axis_name="s")


@functools.partial(
    pl.kernel,
    out_type=jax.ShapeDtypeStruct((_B * _S * _D,), jnp.float32),
    mesh=_mesh,
    scratch_types=[
        pltpu.VMEM((_CW,), jnp.float32),
        pltpu.VMEM((_CW,), jnp.float32),
        pltpu.VMEM((_CW,), jnp.float32),
        pltpu.VMEM((_CW,), jnp.float32),
        pltpu.VMEM((_CW,), jnp.float32),
        pltpu.SemaphoreType.DMA,
        pltpu.SemaphoreType.DMA,
        pltpu.SemaphoreType.DMA,
        pltpu.SemaphoreType.DMA,
    ],
)
def _pos_add(x_hbm, tab_hbm, out_hbm, pos_b, in0, in1, out0, out1,
             ld0, ld1, st0, st1):
    wid = lax.axis_index("s") * _NC + lax.axis_index("c")
    base = wid * (_SPW * _D)
    ins, outs, lds, sts = (in0, in1), (out0, out1), (ld0, ld1), (st0, st1)

    def x_off(c, b):
        return b * _SD + base + c * _CW

    def start_load(c, b, p):
        pltpu.async_copy(x_hbm.at[pl.ds(x_off(c, b), _CW)], ins[p], lds[p])

    def wait_load(p):
        pltpu.make_async_copy(x_hbm.at[pl.ds(0, _CW)], ins[p], lds[p]).wait()

    def start_store(c, b, p):
        pltpu.async_copy(outs[p], out_hbm.at[pl.ds(x_off(c, b), _CW)], sts[p])

    def wait_store(p):
        pltpu.make_async_copy(outs[p], out_hbm.at[pl.ds(0, _CW)], sts[p]).wait()

    def do_add(p):
        src, dst = ins[p], outs[p]

        @plsc.parallel_loop(0, _CW, step=16, unroll=8)
        def add_body(i):
            dst[pl.ds(i, 16)] = src[pl.ds(i, 16)] + pos_b[pl.ds(i, 16)]

    start_load(0, 0, 0)
    start_load(0, 1, 1)
    pltpu.sync_copy(tab_hbm.at[pl.ds(base, _CW)], pos_b)

    for b in range(_B):
        p = b % 2
        wait_load(p)
        if b >= 2:
            wait_store(p)
        do_add(p)
        start_store(0, b, p)
        if b < 2:
            start_load(0, b + 2, p)
        else:
            start_load(1, b - 2, p)

    def chunk_body(c, carry):
        pltpu.sync_copy(tab_hbm.at[pl.ds(base + c * _CW, _CW)], pos_b)
        for b in range(_B):
            p = b % 2
            wait_load(p)
            wait_store(p)
            do_add(p)
            start_store(c, b, p)
            if b < 2:
                start_load(c, b + 2, p)
            else:
                @pl.when(c + 1 < _NCH)
                def _prefetch(c=c, b=b, p=p):
                    start_load(c + 1, b - 2, p)
        return carry

    lax.fori_loop(1, _NCH, chunk_body, 0)
    wait_store(0)
    wait_store(1)


def kernel(x, pos_table):
    out = _pos_add(x.reshape(_B * _S * _D), pos_table.reshape(_S * _D))
    return out.reshape(_B, _S, _D)


# speedup vs baseline: 3.3387x; 2.8084x over previous
import functools

import jax
import jax.numpy as jnp
from jax import lax
from jax.experimental import pallas as pl
from jax.experimental.pallas import tpu as pltpu
from jax.experimental.pallas import tpu_sc as plsc

_B, _S, _D = 4, 8192, 1024
_NC, _NS = 2, 16
_NW = _NC * _NS
_SPW = _S // _NW
_C = 16
_NCH = _SPW // _C
_NJ = _D // 16

_mesh = plsc.VectorSubcoreMesh(core_axis_name="c", subcore_axis_name="s")


@functools.partial(
    pl.kernel,
    out_type=jax.ShapeDtypeStruct((_B, _S, _D), jnp.float32),
    mesh=_mesh,
    scratch_types=[
        pltpu.VMEM((_C, _D), jnp.float32),
        pltpu.VMEM((_C, _D), jnp.float32),
        pltpu.VMEM((_C, _D), jnp.float32),
        pltpu.VMEM((_C, _D), jnp.float32),
        pltpu.VMEM((_C, _D), jnp.float32),
        pltpu.SemaphoreType.DMA,
        pltpu.SemaphoreType.DMA,
        pltpu.SemaphoreType.DMA,
        pltpu.SemaphoreType.DMA,
    ],
)
def _pos_add(x_hbm, tab_hbm, out_hbm, pos_b, in0, in1, out0, out1,
             ld0, ld1, st0, st1):
    wid = lax.axis_index("s") * _NC + lax.axis_index("c")
    s_base = wid * _SPW
    ins, outs, lds, sts = (in0, in1), (out0, out1), (ld0, ld1), (st0, st1)

    def s0(c):
        return s_base + c * _C

    def start_load(c, b, p):
        pltpu.async_copy(x_hbm.at[b, pl.ds(s0(c), _C)], ins[p], lds[p])

    def wait_load(p):
        pltpu.make_async_copy(x_hbm.at[0, pl.ds(0, _C)], ins[p], lds[p]).wait()

    def start_store(c, b, p):
        pltpu.async_copy(outs[p], out_hbm.at[b, pl.ds(s0(c), _C)], sts[p])

    def wait_store(p):
        pltpu.make_async_copy(outs[p], out_hbm.at[0, pl.ds(0, _C)], sts[p]).wait()

    def do_add(p):
        src, dst = ins[p], outs[p]

        @plsc.parallel_loop(0, _C * _NJ, unroll=4)
        def add_vec(i):
            r = i >> 6
            j = (i & (_NJ - 1)) * 16
            dst[r, pl.ds(j, 16)] = src[r, pl.ds(j, 16)] + pos_b[r, pl.ds(j, 16)]

    start_load(0, 0, 0)
    start_load(0, 1, 1)
    pltpu.sync_copy(tab_hbm.at[pl.ds(s0(0), _C)], pos_b)

    for b in range(_B):
        p = b % 2
        wait_load(p)
        if b >= 2:
            wait_store(p)
        do_add(p)
        start_store(0, b, p)
        if b < 2:
            start_load(0, b + 2, p)
        else:
            start_load(1, b - 2, p)

    def chunk_body(c, carry):
        pltpu.sync_copy(tab_hbm.at[pl.ds(s0(c), _C)], pos_b)
        for b in range(_B):
            p = b % 2
            wait_load(p)
            wait_store(p)
            do_add(p)
            start_store(c, b, p)
            if b < 2:
                start_load(c, b + 2, p)
            else:
                @pl.when(c + 1 < _NCH)
                def _prefetch(c=c, b=b, p=p):
                    start_load(c + 1, b - 2, p)
        return carry

    lax.fori_loop(1, _NCH, chunk_body, 0)
    wait_store(0)
    wait_store(1)


def kernel(x, pos_table):
    return _pos_add(x, pos_table)


# speedup vs baseline: 3.8734x; 1.1601x over previous
import functools

import jax
import jax.numpy as jnp
from jax import lax
from jax.experimental import pallas as pl
from jax.experimental.pallas import tpu as pltpu
from jax.experimental.pallas import tpu_sc as plsc

_B, _S, _D = 4, 8192, 1024
_NC, _NS = 2, 16
_NW = _NC * _NS
_SPW = _S // _NW
_C = 8
_NCH = _SPW // _C
_NV = _C * _D // 16
_NB = 8

_mesh = plsc.VectorSubcoreMesh(core_axis_name="c", subcore_axis_name="s")

_scratch = (
    [pltpu.VMEM((_C, _D), jnp.float32)] * (_NB + 2)
    + [pltpu.SemaphoreType.DMA] * (2 * _NB + 2)
)


@functools.partial(
    pl.kernel,
    out_type=jax.ShapeDtypeStruct((_B, _S, _D), jnp.float32),
    mesh=_mesh,
    scratch_types=_scratch,
)
def _pos_add(x_hbm, tab_hbm, out_hbm, *scr):
    xb = scr[:_NB]
    pb = scr[_NB:_NB + 2]
    ld = scr[_NB + 2:2 * _NB + 2]
    st = scr[2 * _NB + 2:3 * _NB + 2]
    ps = scr[3 * _NB + 2:]

    wid = lax.axis_index("s") * _NC + lax.axis_index("c")
    s_base = wid * _SPW

    def s0(c):
        return s_base + c * _C

    def start_load(c, b, k):
        pltpu.async_copy(x_hbm.at[b, pl.ds(s0(c), _C)], xb[k], ld[k])

    def wait_load(k):
        pltpu.make_async_copy(x_hbm.at[0, pl.ds(0, _C)], xb[k], ld[k]).wait()

    def start_store(c, b, k):
        pltpu.async_copy(xb[k], out_hbm.at[b, pl.ds(s0(c), _C)], st[k])

    def wait_store(k):
        pltpu.make_async_copy(xb[k], out_hbm.at[0, pl.ds(0, _C)], st[k]).wait()

    def start_pos(c, q):
        pltpu.async_copy(tab_hbm.at[pl.ds(s0(c), _C)], pb[q], ps[q])

    def wait_pos(q):
        pltpu.make_async_copy(tab_hbm.at[pl.ds(0, _C)], pb[q], ps[q]).wait()

    def do_add(k, q):
        buf, pos = xb[k], pb[q]

        @plsc.parallel_loop(0, _NV, unroll=4)
        def add_vec(i):
            r = i >> 6
            j = (i & 63) * 16
            plsc.addupdate(buf.at[r, pl.ds(j, 16)], pos[r, pl.ds(j, 16)])

    def gen_iter(c, cc, b, peeled_first):
        k = 4 * cc + b
        wait_load(k)
        do_add(k, cc)
        start_store(c, b, k)
        k2 = (k + 6) % 8
        cp, bp = (c + 1, b + 2) if b < 2 else (c + 2, b - 2)

        def issue():
            if not peeled_first:
                wait_store(k2)
            start_load(cp, bp, k2)

        if peeled_first or isinstance(cp, int):
            issue()
        else:
            pl.when(cp < _NCH)(issue)

    for b in range(_B):
        start_load(0, b, b)
    start_load(1, 0, 4)
    start_load(1, 1, 5)
    start_pos(0, 0)
    start_pos(1, 1)

    for c in (0, 1):
        wait_pos(c)
        for b in range(_B):
            gen_iter(c, c, b, peeled_first=(c == 0 and b < 2))
        start_pos(c + 2, c)

    def chunk_pair(c2, carry):
        for cc in (0, 1):
            c = 2 * c2 + cc
            wait_pos(cc)
            for b in range(_B):
                gen_iter(c, cc, b, peeled_first=False)

            @pl.when(c + 2 < _NCH)
            def _pos_prefetch(c=c, cc=cc):
                start_pos(c + 2, cc)
        return carry

    lax.fori_loop(1, _NCH // 2, chunk_pair, 0)

    for k in range(_NB):
        wait_store(k)


def kernel(x, pos_table):
    return _pos_add(x, pos_table)
